# Initial kernel scaffold; baseline (speedup 1.0000x reference)
import jax
import jax.numpy as jnp
from jax.experimental import pallas as pl


def kernel(inputs, adj, w1, w2):
    raise NotImplementedError("write your pallas kernel here")


# speedup vs baseline: 7.0222x; 7.0222x over previous
import jax
import jax.numpy as jnp
from jax.experimental import pallas as pl
from jax.experimental.pallas import tpu as pltpu

N = 512
D = 128
H = 64
BI = 8
BK = 128


def _pre_kernel(z_ref, adj_ref, w1_ref, x_ref, c_ref, b_ref):
    z = z_ref[...]
    x_ref[...] = jax.lax.dot_general(
        z, z, (((1,), (1,)), ((), ())), preferred_element_type=jnp.float32)
    row = jax.lax.broadcasted_iota(jnp.int32, (N, N), 0)
    col = jax.lax.broadcasted_iota(jnp.int32, (N, N), 1)
    ltri = (col < row).astype(jnp.float32)
    c_ref[...] = jnp.dot(ltri, adj_ref[...], preferred_element_type=jnp.float32)
    b_ref[...] = jnp.dot(z, w1_ref[0:D, :], preferred_element_type=jnp.float32)


def _main_kernel(adj_ref, c_ref, b_ref, w2r_ref, wt_ref, g_scr):
    p = pl.program_id(0)
    i0 = p * BI
    cblk = c_ref[pl.ds(i0, BI), :]
    rowb = jax.lax.broadcasted_iota(jnp.int32, (BI, N), 0)
    colj = jax.lax.broadcasted_iota(jnp.int32, (BI, N), 1)
    mask = (colj < (i0 + rowb)).astype(jnp.float32)
    u_rows = mask * jax.lax.rsqrt(jnp.maximum(cblk, 1.0))
    u_cols = jnp.transpose(u_rows)

    b_mat = b_ref[...]
    g_scr[...] = jnp.concatenate(
        [u_cols[:, b:b + 1] * b_mat for b in range(BI)], axis=1)

    nk = (i0 + BI - 1 + BK - 1) // BK

    def body(kb, acc):
        ablk = adj_ref[:, pl.ds(kb * BK, BK)]
        gblk = g_scr[pl.ds(kb * BK, BK), :]
        return acc + jnp.dot(ablk, gblk, preferred_element_type=jnp.float32)

    t = jax.lax.fori_loop(
        0, nk, body, jnp.zeros((N, BI * H), dtype=jnp.float32))

    w2r = w2r_ref[...]
    w_cols = []
    for b in range(BI):
        tb = t[:, b * H:(b + 1) * H]
        vb = jnp.sum(jnp.maximum(tb, 0.0) * w2r, axis=1, keepdims=True)
        ub = u_cols[:, b:b + 1]
        w_cols.append(ub * ub * vb)
    wt_ref[...] = jnp.transpose(jnp.concatenate(w_cols, axis=1))


def _final_kernel(x_ref, adj_ref, wt_ref, c_ref, out_ref):
    a2 = jax.lax.dot_general(
        adj_ref[...], wt_ref[...], (((1,), (1,)), ((), ())),
        preferred_element_type=jnp.float32)
    ct = jnp.transpose(c_ref[...])
    row = jax.lax.broadcasted_iota(jnp.int32, (N, N), 0)
    col = jax.lax.broadcasted_iota(jnp.int32, (N, N), 1)
    mask = (row < col).astype(jnp.float32)
    g = mask * jax.lax.rsqrt(jnp.maximum(ct, 1.0)) * a2
    out_ref[...] = x_ref[...] + 0.5 * (g + jnp.transpose(g))


def kernel(inputs, adj, w1, w2):
    z = inputs
    f32 = jnp.float32
    x, c, b = pl.pallas_call(
        _pre_kernel,
        out_shape=[
            jax.ShapeDtypeStruct((N, N), f32),
            jax.ShapeDtypeStruct((N, N), f32),
            jax.ShapeDtypeStruct((N, H), f32),
        ],
    )(z, adj, w1)

    w2r = w2.reshape(1, H)
    wt = pl.pallas_call(
        _main_kernel,
        grid=(N // BI,),
        in_specs=[
            pl.BlockSpec((N, N), lambda p: (0, 0)),
            pl.BlockSpec((N, N), lambda p: (0, 0)),
            pl.BlockSpec((N, H), lambda p: (0, 0)),
            pl.BlockSpec((1, H), lambda p: (0, 0)),
        ],
        out_specs=pl.BlockSpec((BI, N), lambda p: (p, 0)),
        out_shape=jax.ShapeDtypeStruct((N, N), f32),
        scratch_shapes=[pltpu.VMEM((N, BI * H), f32)],
    )(adj, c, b, w2r)

    out = pl.pallas_call(
        _final_kernel,
        out_shape=jax.ShapeDtypeStruct((N, N), f32),
    )(x, adj, wt, c)
    return out


# speedup vs baseline: 7.1582x; 1.0194x over previous
import jax
import jax.numpy as jnp
from jax.experimental import pallas as pl
from jax.experimental.pallas import tpu as pltpu

N = 512
D = 128
H = 64
BI = 8
BK = 128


def _pre_kernel(z_ref, adj_ref, w1_ref, x_ref, c_ref, b_ref, adjbf_ref):
    z = z_ref[...]
    x_ref[...] = jax.lax.dot_general(
        z, z, (((1,), (1,)), ((), ())), preferred_element_type=jnp.float32)
    row = jax.lax.broadcasted_iota(jnp.int32, (N, N), 0)
    col = jax.lax.broadcasted_iota(jnp.int32, (N, N), 1)
    ltri = (col < row).astype(jnp.float32)
    c_ref[...] = jnp.dot(ltri, adj_ref[...], preferred_element_type=jnp.float32)
    b_ref[...] = jnp.dot(z, w1_ref[0:D, :], preferred_element_type=jnp.float32)
    adjbf_ref[...] = adj_ref[...].astype(jnp.bfloat16)


def _main_kernel(adj_ref, c_ref, b_ref, w2r_ref, wt_ref, g_scr):
    p = pl.program_id(0)
    i0 = p * BI
    cblk = c_ref[pl.ds(i0, BI), :]
    rowb = jax.lax.broadcasted_iota(jnp.int32, (BI, N), 0)
    colj = jax.lax.broadcasted_iota(jnp.int32, (BI, N), 1)
    mask = (colj < (i0 + rowb)).astype(jnp.float32)
    u_rows = mask * jax.lax.rsqrt(jnp.maximum(cblk, 1.0))
    u_cols = jnp.transpose(u_rows)

    b_mat = b_ref[...]
    g_scr[...] = jnp.concatenate(
        [u_cols[:, b:b + 1] * b_mat for b in range(BI)],
        axis=1).astype(jnp.bfloat16)

    nk = (i0 + BI - 1 + BK - 1) // BK

    def body(kb, acc):
        ablk = adj_ref[:, pl.ds(kb * BK, BK)]
        gblk = g_scr[pl.ds(kb * BK, BK), :]
        return acc + jnp.dot(ablk, gblk, preferred_element_type=jnp.float32)

    t = jax.lax.fori_loop(
        0, nk, body, jnp.zeros((N, BI * H), dtype=jnp.float32))

    w2r = w2r_ref[...]
    w_cols = []
    for b in range(BI):
        tb = t[:, b * H:(b + 1) * H]
        vb = jnp.sum(jnp.maximum(tb, 0.0) * w2r, axis=1, keepdims=True)
        ub = u_cols[:, b:b + 1]
        w_cols.append(ub * ub * vb)
    wt_ref[...] = jnp.transpose(jnp.concatenate(w_cols, axis=1))


def _final_kernel(x_ref, adj_ref, wt_ref, c_ref, out_ref):
    a2 = jax.lax.dot_general(
        adj_ref[...], wt_ref[...], (((1,), (1,)), ((), ())),
        preferred_element_type=jnp.float32)
    ct = jnp.transpose(c_ref[...])
    row = jax.lax.broadcasted_iota(jnp.int32, (N, N), 0)
    col = jax.lax.broadcasted_iota(jnp.int32, (N, N), 1)
    mask = (row < col).astype(jnp.float32)
    g = mask * jax.lax.rsqrt(jnp.maximum(ct, 1.0)) * a2
    out_ref[...] = x_ref[...] + 0.5 * (g + jnp.transpose(g))


def kernel(inputs, adj, w1, w2):
    z = inputs
    f32 = jnp.float32
    x, c, b, adjbf = pl.pallas_call(
        _pre_kernel,
        out_shape=[
            jax.ShapeDtypeStruct((N, N), f32),
            jax.ShapeDtypeStruct((N, N), f32),
            jax.ShapeDtypeStruct((N, H), f32),
            jax.ShapeDtypeStruct((N, N), jnp.bfloat16),
        ],
    )(z, adj, w1)

    w2r = w2.reshape(1, H)
    wt = pl.pallas_call(
        _main_kernel,
        grid=(N // BI,),
        in_specs=[
            pl.BlockSpec((N, N), lambda p: (0, 0)),
            pl.BlockSpec((N, N), lambda p: (0, 0)),
            pl.BlockSpec((N, H), lambda p: (0, 0)),
            pl.BlockSpec((1, H), lambda p: (0, 0)),
        ],
        out_specs=pl.BlockSpec((BI, N), lambda p: (p, 0)),
        out_shape=jax.ShapeDtypeStruct((N, N), f32),
        scratch_shapes=[pltpu.VMEM((N, BI * H), jnp.bfloat16)],
    )(adjbf, c, b, w2r)

    out = pl.pallas_call(
        _final_kernel,
        out_shape=jax.ShapeDtypeStruct((N, N), f32),
    )(x, adj, wt, c)
    return out


# speedup vs baseline: 40.9702x; 5.7235x over previous
import jax
import jax.numpy as jnp
from jax.experimental import pallas as pl

N = 512
D = 128
H = 64
BI = 8


def _fused_kernel(z_ref, adj_ref, w1_ref, w2r_ref, out_ref):
    f32 = jnp.float32
    z = z_ref[...]
    adj = adj_ref[...]
    x = jax.lax.dot_general(
        z, z, (((1,), (1,)), ((), ())), preferred_element_type=f32)
    row = jax.lax.broadcasted_iota(jnp.int32, (N, N), 0)
    col = jax.lax.broadcasted_iota(jnp.int32, (N, N), 1)
    ltri = (col < row).astype(f32)
    c = jnp.dot(ltri, adj, preferred_element_type=f32)
    b_mat = jnp.dot(z, w1_ref[0:D, :], preferred_element_type=f32)
    b8 = jnp.concatenate([b_mat] * BI, axis=1)
    w2t = jnp.concatenate([w2r_ref[...]] * BI, axis=1)
    adjbf = adj.astype(jnp.bfloat16)

    rowb = jax.lax.broadcasted_iota(jnp.int32, (BI, N), 0)
    colj = jax.lax.broadcasted_iota(jnp.int32, (BI, N), 1)
    rowe = jax.lax.broadcasted_iota(jnp.int32, (BI, BI * H), 0)
    cole = jax.lax.broadcasted_iota(jnp.int32, (BI, BI * H), 1)
    sel = ((cole // H) == rowe).astype(f32)

    wt_rows = []
    for s in range(N // BI):
        i0 = s * BI
        r_hi = i0 + BI
        cblk = c[i0:i0 + BI, :]
        mask = (colj < (i0 + rowb)).astype(f32)
        u_rows = mask * jax.lax.rsqrt(jnp.maximum(cblk, 1.0))
        u_wide = jax.lax.dot_general(
            u_rows, sel, (((0,), (0,)), ((), ())),
            preferred_element_type=f32)
        gcat = (u_wide * b8).astype(jnp.bfloat16)
        t = jnp.dot(adjbf[0:r_hi, :], gcat,
                    preferred_element_type=f32)
        m = jnp.maximum(t, 0.0) * w2t
        v = jax.lax.dot_general(
            sel, m, (((1,), (1,)), ((), ())),
            preferred_element_type=f32)
        if r_hi < N:
            v = jnp.concatenate(
                [v, jnp.zeros((BI, N - r_hi), dtype=f32)], axis=1)
        wt_rows.append(u_rows * u_rows * v)
    wt = jnp.concatenate(wt_rows, axis=0)

    a2 = jax.lax.dot_general(
        adj, wt, (((1,), (1,)), ((), ())), preferred_element_type=f32)
    ct = jnp.transpose(c)
    maskf = (row < col).astype(f32)
    g = maskf * jax.lax.rsqrt(jnp.maximum(ct, 1.0)) * a2
    out_ref[...] = x + 0.5 * (g + jnp.transpose(g))


def kernel(inputs, adj, w1, w2):
    w2r = w2.reshape(1, H)
    out = pl.pallas_call(
        _fused_kernel,
        out_shape=jax.ShapeDtypeStruct((N, N), jnp.float32),
    )(inputs, adj, w1, w2r)
    return out


# speedup vs baseline: 44.5362x; 1.0870x over previous
import jax
import jax.numpy as jnp
from jax.experimental import pallas as pl

N = 512
D = 128
H = 64
BI = 8


def _fused_kernel(z_ref, adj_ref, w1_ref, w2r_ref, out_ref):
    f32 = jnp.float32
    z = z_ref[...]
    adj = adj_ref[...]
    x = jax.lax.dot_general(
        z, z, (((1,), (1,)), ((), ())), preferred_element_type=f32)
    row = jax.lax.broadcasted_iota(jnp.int32, (N, N), 0)
    col = jax.lax.broadcasted_iota(jnp.int32, (N, N), 1)
    ltri = (col < row).astype(f32)
    c = jnp.dot(ltri, adj, preferred_element_type=f32)
    b_mat = jnp.dot(z, w1_ref[0:D, :], preferred_element_type=f32)
    b8 = jnp.concatenate([b_mat] * BI, axis=1)
    w2t = jnp.concatenate([w2r_ref[...]] * BI, axis=1)
    adjbf = adj.astype(jnp.bfloat16)

    rowb = jax.lax.broadcasted_iota(jnp.int32, (BI, N), 0)
    colj = jax.lax.broadcasted_iota(jnp.int32, (BI, N), 1)
    rowe = jax.lax.broadcasted_iota(jnp.int32, (BI, BI * H), 0)
    cole = jax.lax.broadcasted_iota(jnp.int32, (BI, BI * H), 1)
    sel = ((cole // H) == rowe).astype(f32)

    wt_rows = []
    for s in range(N // BI):
        i0 = s * BI
        r_hi = i0 + BI
        cblk = c[i0:i0 + BI, :]
        mask = (colj < (i0 + rowb)).astype(f32)
        u_rows = mask * jax.lax.rsqrt(jnp.maximum(cblk, 1.0))
        k_hi = min(N, ((r_hi + 127) // 128) * 128)
        u_wide = jax.lax.dot_general(
            u_rows[:, 0:k_hi], sel, (((0,), (0,)), ((), ())),
            preferred_element_type=f32)
        gcat = (u_wide * b8[0:k_hi, :]).astype(jnp.bfloat16)
        t = jnp.dot(adjbf[0:r_hi, 0:k_hi], gcat,
                    preferred_element_type=f32)
        m = jnp.maximum(t, 0.0) * w2t
        v = jax.lax.dot_general(
            sel, m, (((1,), (1,)), ((), ())),
            preferred_element_type=f32)
        if r_hi < N:
            v = jnp.concatenate(
                [v, jnp.zeros((BI, N - r_hi), dtype=f32)], axis=1)
        wt_rows.append(u_rows * u_rows * v)
    wt = jnp.concatenate(wt_rows, axis=0)

    a2 = jax.lax.dot_general(
        adj, wt, (((1,), (1,)), ((), ())), preferred_element_type=f32)
    ct = jnp.transpose(c)
    maskf = (row < col).astype(f32)
    g = maskf * jax.lax.rsqrt(jnp.maximum(ct, 1.0)) * a2
    out_ref[...] = x + 0.5 * (g + jnp.transpose(g))


def kernel(inputs, adj, w1, w2):
    w2r = w2.reshape(1, H)
    out = pl.pallas_call(
        _fused_kernel,
        out_shape=jax.ShapeDtypeStruct((N, N), jnp.float32),
    )(inputs, adj, w1, w2r)
    return out


# speedup vs baseline: 61.8690x; 1.3892x over previous
import jax
import jax.numpy as jnp
from jax.experimental import pallas as pl

N = 512
D = 128
H = 64
BI = 16


def _fused_kernel(z_ref, adj_ref, w1_ref, w2r_ref, out_ref):
    f32 = jnp.float32
    z = z_ref[...]
    adj = adj_ref[...]
    x = jax.lax.dot_general(
        z, z, (((1,), (1,)), ((), ())), preferred_element_type=f32)
    row = jax.lax.broadcasted_iota(jnp.int32, (N, N), 0)
    col = jax.lax.broadcasted_iota(jnp.int32, (N, N), 1)
    ltri = (col < row).astype(f32)
    c = jnp.dot(ltri, adj, preferred_element_type=f32)
    b_mat = jnp.dot(z, w1_ref[0:D, :], preferred_element_type=f32)
    b8 = jnp.concatenate([b_mat] * BI, axis=1)
    w2t = jnp.concatenate([w2r_ref[...]] * BI, axis=1)
    adjbf = adj.astype(jnp.bfloat16)

    rowb = jax.lax.broadcasted_iota(jnp.int32, (BI, N), 0)
    colj = jax.lax.broadcasted_iota(jnp.int32, (BI, N), 1)
    rowe = jax.lax.broadcasted_iota(jnp.int32, (BI, BI * H), 0)
    cole = jax.lax.broadcasted_iota(jnp.int32, (BI, BI * H), 1)
    sel = ((cole // H) == rowe).astype(f32)

    wt_rows = []
    for s in range(N // BI):
        i0 = s * BI
        r_hi = i0 + BI
        cblk = c[i0:i0 + BI, :]
        mask = (colj < (i0 + rowb)).astype(f32)
        u_rows = mask * jax.lax.rsqrt(jnp.maximum(cblk, 1.0))
        k_hi = min(N, ((r_hi + 127) // 128) * 128)
        u_wide = jax.lax.dot_general(
            u_rows[:, 0:k_hi], sel, (((0,), (0,)), ((), ())),
            preferred_element_type=f32)
        gcat = (u_wide * b8[0:k_hi, :]).astype(jnp.bfloat16)
        t = jnp.dot(adjbf[0:r_hi, 0:k_hi], gcat,
                    preferred_element_type=f32)
        m = jnp.maximum(t, 0.0) * w2t
        v = jax.lax.dot_general(
            sel, m, (((1,), (1,)), ((), ())),
            preferred_element_type=f32)
        if r_hi < N:
            v = jnp.concatenate(
                [v, jnp.zeros((BI, N - r_hi), dtype=f32)], axis=1)
        wt_rows.append(u_rows * u_rows * v)
    wt = jnp.concatenate(wt_rows, axis=0)

    a2 = jax.lax.dot_general(
        adj, wt, (((1,), (1,)), ((), ())), preferred_element_type=f32)
    ct = jnp.transpose(c)
    maskf = (row < col).astype(f32)
    g = maskf * jax.lax.rsqrt(jnp.maximum(ct, 1.0)) * a2
    out_ref[...] = x + 0.5 * (g + jnp.transpose(g))


def kernel(inputs, adj, w1, w2):
    w2r = w2.reshape(1, H)
    out = pl.pallas_call(
        _fused_kernel,
        out_shape=jax.ShapeDtypeStruct((N, N), jnp.float32),
    )(inputs, adj, w1, w2r)
    return out
